# baseline (speedup 1.0000x reference)
import jax
import jax.numpy as jnp
from jax import lax
from jax.experimental import pallas as pl
from jax.experimental.pallas import tpu as pltpu
from jax.experimental.pallas import tpu_sc as plsc

VOCAB = 1_000_000
EMB = 64
RANK = 1
WORLD = 4
LOCAL_N = VOCAB // WORLD
OFFSET = RANK * LOCAL_N
BATCH = 4096
SEQ = 50
TOTAL = BATCH * SEQ

NC = 2
NS = 16
NW = NC * NS

PER_W = TOTAL // NW
CHUNK = 640
NCHUNK = PER_W // CHUNK
NBUF = 2
SG = 16
MAXS = CHUNK // SG
NGRP = CHUNK // 16
CROW = CHUNK + 16


def _tec_body(ids_hbm, table_hbm, out_hbm, ids_v, cid_flat, cid2d, cpos_flat,
              czer_flat, gbuf, gsem0, gsem1, ssem):
    gsems = (gsem0, gsem1)
    wid = lax.axis_index("s") * NC + lax.axis_index("c")
    base = wid * PER_W

    zi = jnp.zeros((16,), jnp.int32)
    zf = jnp.zeros((16,), jnp.float32)
    of = jnp.ones((16,), jnp.float32)

    pltpu.sync_copy(ids_hbm.at[pl.ds(base, PER_W)], ids_v)

    for t in range(0, NBUF * CROW, 16):
        cid_flat[pl.ds(t, 16)] = zi
    for b in range(NBUF):
        for k in range(4):
            gbuf[b, 0, pl.ds(16 * k, 16)] = zf

    iota16 = lax.iota(jnp.int32, 16)

    def compute_chunk(g):
        b = g % NBUF

        def jbody(j, carry):
            cnt, zcnt = carry
            v = ids_v[pl.ds(g * CHUNK + 16 * j, 16)]
            adj = v - OFFSET
            valid = (adj >= 0) & (adj < LOCAL_N)
            pos = iota16 + (16 * j)
            plsc.store_compressed(
                cid_flat.at[pl.ds(b * CROW + cnt, 16)], adj,
                mask=valid)
            plsc.store_compressed(
                cpos_flat.at[pl.ds(b * CROW + cnt, 16)], pos,
                mask=valid)
            plsc.store_compressed(
                czer_flat.at[pl.ds(b * CROW + zcnt, 16)], pos,
                mask=jnp.logical_not(valid))
            nv = plsc.cumsum(jnp.where(valid, jnp.full((16,), 1, jnp.int32),
                                       zi))[15]
            return (cnt + nv, zcnt + (16 - nv))

        cnt, zcnt = lax.fori_loop(0, NGRP, jbody,
                                  (jnp.int32(0), jnp.int32(0)))
        sent = jnp.full((16,), CHUNK, jnp.int32)
        cpos_flat[pl.ds(b * CROW + cnt, 16)] = sent
        czer_flat[pl.ds(b * CROW + zcnt, 16)] = sent
        return (cnt, zcnt)

    def fire_gather(g, cnt):
        b = g % NBUF
        ns = (cnt + (SG - 1)) // SG

        def sbody(s, _):
            iv = cid_flat[pl.ds(b * CROW + s * SG, 16)]
            pltpu.async_copy(table_hbm.at[iv],
                             gbuf.at[b, pl.ds(s * SG, SG), :], gsems[b])
            return 0

        lax.fori_loop(0, ns, sbody, 0)

    def wait_gather(g, cnt):
        b = g % NBUF
        ns = (cnt + (SG - 1)) // SG

        def wbody(s, _):
            pltpu.make_async_copy(table_hbm.at[cid2d.at[b, 0]],
                                  gbuf.at[b, pl.ds(0, SG), :], gsems[b]).wait()
            return 0

        lax.fori_loop(0, ns, wbody, 0)

    def expand(g, cnt, zcnt):
        b = g % NBUF
        nv16 = (cnt + 15) // 16
        nz16 = (zcnt + 15) // 16

        def pbody(t, _):
            grp = nv16 - 1 - t
            dvec = cpos_flat[pl.ds(b * CROW + 16 * grp, 16)]
            for jj in range(15, -1, -1):
                dst = dvec[jj]
                src16 = 16 * grp + jj
                for k in range(4):
                    r = gbuf[b, src16, pl.ds(16 * k, 16)]
                    gbuf[b, dst, pl.ds(16 * k, 16)] = r
            return 0

        lax.fori_loop(0, nv16, pbody, 0)

        def zbody(t, _):
            zvec = czer_flat[pl.ds(b * CROW + 16 * t, 16)]
            for jj in range(16):
                dst = zvec[jj]
                for k in range(4):
                    gbuf[b, dst, pl.ds(16 * k, 16)] = zf
            return 0

        lax.fori_loop(0, nz16, zbody, 0)

    def fire_scatter(g):
        b = g % NBUF
        pltpu.async_copy(
            gbuf.at[b, pl.ds(0, CHUNK), :],
            out_hbm.at[pl.ds(base + g * CHUNK, CHUNK), :],
            ssem)

    def drain_scatter():
        pltpu.make_async_copy(
            gbuf.at[0, pl.ds(0, CHUNK), :],
            out_hbm.at[pl.ds(base, CHUNK), :],
            ssem).wait()

    cnts = [None] * NCHUNK
    cnts[0] = compute_chunk(0)
    fire_gather(0, cnts[0][0])
    for g in range(NCHUNK):
        if g + 1 < NCHUNK:
            cnts[g + 1] = compute_chunk(g + 1)
            if g >= 1:
                drain_scatter()
            fire_gather(g + 1, cnts[g + 1][0])
        wait_gather(g, cnts[g][0])
        expand(g, cnts[g][0], cnts[g][1])
        fire_scatter(g)
    drain_scatter()
    drain_scatter()


@jax.jit
def _embed(ids_flat, weights):
    kern = pl.kernel(
        _tec_body,
        out_type=jax.ShapeDtypeStruct((TOTAL, EMB), jnp.float32),
        mesh=plsc.VectorSubcoreMesh(core_axis_name="c", subcore_axis_name="s"),
        scratch_types=[
            pltpu.VMEM((PER_W,), jnp.int32),
            pltpu.VMEM((NBUF * CROW,), jnp.int32),
            pltpu.VMEM((NBUF, MAXS, SG), jnp.int32),
            pltpu.VMEM((NBUF * CROW,), jnp.int32),
            pltpu.VMEM((NBUF * CROW,), jnp.int32),
            pltpu.VMEM((NBUF, CHUNK + 1, EMB), jnp.float32),
            pltpu.SemaphoreType.DMA,
            pltpu.SemaphoreType.DMA,
            pltpu.SemaphoreType.DMA,
        ],
        compiler_params=pltpu.CompilerParams(use_tc_tiling_on_sc=False,
                                             needs_layout_passes=False),
    )
    return kern(ids_flat, weights)


def kernel(input_ids, weights):
    out = _embed(input_ids.reshape(TOTAL), weights)
    return out.reshape(BATCH, SEQ, EMB)


# speedup vs baseline: 1.0140x; 1.0140x over previous
import jax
import jax.numpy as jnp
from jax import lax
from jax.experimental import pallas as pl
from jax.experimental.pallas import tpu as pltpu
from jax.experimental.pallas import tpu_sc as plsc

VOCAB = 1_000_000
EMB = 64
RANK = 1
WORLD = 4
LOCAL_N = VOCAB // WORLD
OFFSET = RANK * LOCAL_N
BATCH = 4096
SEQ = 50
TOTAL = BATCH * SEQ

NC = 2
NS = 16
NW = NC * NS

PER_W = TOTAL // NW
CHUNK = 800
NCHUNK = PER_W // CHUNK
NBUF = 2
SG = 16
MAXS = CHUNK // SG
NGRP = CHUNK // 16
CROW = CHUNK + 16


def _tec_body(ids_hbm, table_hbm, out_hbm, ids_v, cid_flat, cpos_flat,
              czer_flat, gbuf, gsem0, gsem1, ssem):
    gsems = (gsem0, gsem1)
    wid = lax.axis_index("s") * NC + lax.axis_index("c")
    base = wid * PER_W

    zi = jnp.zeros((16,), jnp.int32)
    zf = jnp.zeros((16,), jnp.float32)
    of = jnp.ones((16,), jnp.float32)

    pltpu.sync_copy(ids_hbm.at[pl.ds(base, PER_W)], ids_v)

    for t in range(0, NBUF * CROW, 16):
        cid_flat[pl.ds(t, 16)] = zi
    for b in range(NBUF):
        for k in range(4):
            gbuf[b, 0, pl.ds(16 * k, 16)] = zf

    iota16 = lax.iota(jnp.int32, 16)

    def compute_chunk(g):
        b = g % NBUF

        def jbody(j, carry):
            cnt, zcnt = carry
            v = ids_v[pl.ds(g * CHUNK + 16 * j, 16)]
            adj = v - OFFSET
            valid = (adj >= 0) & (adj < LOCAL_N)
            pos = iota16 + (16 * j)
            plsc.store_compressed(
                cid_flat.at[pl.ds(b * CROW + cnt, 16)], adj,
                mask=valid)
            plsc.store_compressed(
                cpos_flat.at[pl.ds(b * CROW + cnt, 16)], pos,
                mask=valid)
            plsc.store_compressed(
                czer_flat.at[pl.ds(b * CROW + zcnt, 16)], pos,
                mask=jnp.logical_not(valid))
            nv = plsc.cumsum(jnp.where(valid, jnp.full((16,), 1, jnp.int32),
                                       zi))[15]
            return (cnt + nv, zcnt + (16 - nv))

        cnt, zcnt = lax.fori_loop(0, NGRP, jbody,
                                  (jnp.int32(0), jnp.int32(0)))
        sent = jnp.full((16,), CHUNK, jnp.int32)
        cpos_flat[pl.ds(b * CROW + cnt, 16)] = sent
        czer_flat[pl.ds(b * CROW + zcnt, 16)] = sent
        return (cnt, zcnt)

    def fire_gather(g, cnt):
        b = g % NBUF
        ns = (cnt + (SG - 1)) // SG

        def sbody(s, _):
            iv = cid_flat[pl.ds(b * CROW + s * SG, 16)]
            pltpu.async_copy(table_hbm.at[iv],
                             gbuf.at[b, pl.ds(s * SG, SG), :], gsems[b])
            return 0

        lax.fori_loop(0, ns, sbody, 0)

    def wait_gather(g, cnt):
        b = g % NBUF
        ns = (cnt + (SG - 1)) // SG

        def wbody(s, _):
            iv = cid_flat[pl.ds(b * CROW, 16)]
            pltpu.make_async_copy(table_hbm.at[iv],
                                  gbuf.at[b, pl.ds(0, SG), :], gsems[b]).wait()
            return 0

        lax.fori_loop(0, ns, wbody, 0)

    def expand(g, cnt, zcnt):
        b = g % NBUF
        nv16 = (cnt + 15) // 16
        nz16 = (zcnt + 15) // 16

        def pbody(t, _):
            grp = nv16 - 1 - t
            dvec = cpos_flat[pl.ds(b * CROW + 16 * grp, 16)]
            for jj in range(15, -1, -1):
                dst = dvec[jj]
                src16 = 16 * grp + jj
                for k in range(4):
                    r = gbuf[b, src16, pl.ds(16 * k, 16)]
                    gbuf[b, dst, pl.ds(16 * k, 16)] = r
            return 0

        lax.fori_loop(0, nv16, pbody, 0)

        def zbody(t, _):
            zvec = czer_flat[pl.ds(b * CROW + 16 * t, 16)]
            for jj in range(16):
                dst = zvec[jj]
                for k in range(4):
                    gbuf[b, dst, pl.ds(16 * k, 16)] = zf
            return 0

        lax.fori_loop(0, nz16, zbody, 0)

    def fire_scatter(g):
        b = g % NBUF
        pltpu.async_copy(
            gbuf.at[b, pl.ds(0, CHUNK), :],
            out_hbm.at[pl.ds(base + g * CHUNK, CHUNK), :],
            ssem)

    def drain_scatter():
        pltpu.make_async_copy(
            gbuf.at[0, pl.ds(0, CHUNK), :],
            out_hbm.at[pl.ds(base, CHUNK), :],
            ssem).wait()

    cnts = [None] * NCHUNK
    cnts[0] = compute_chunk(0)
    fire_gather(0, cnts[0][0])
    for g in range(NCHUNK):
        if g + 1 < NCHUNK:
            cnts[g + 1] = compute_chunk(g + 1)
            if g >= 1:
                drain_scatter()
            fire_gather(g + 1, cnts[g + 1][0])
        wait_gather(g, cnts[g][0])
        expand(g, cnts[g][0], cnts[g][1])
        fire_scatter(g)
    drain_scatter()
    drain_scatter()


@jax.jit
def _embed(ids_flat, weights):
    kern = pl.kernel(
        _tec_body,
        out_type=jax.ShapeDtypeStruct((TOTAL, EMB), jnp.float32),
        mesh=plsc.VectorSubcoreMesh(core_axis_name="c", subcore_axis_name="s"),
        scratch_types=[
            pltpu.VMEM((PER_W,), jnp.int32),
            pltpu.VMEM((NBUF * CROW,), jnp.int32),
            pltpu.VMEM((NBUF * CROW,), jnp.int32),
            pltpu.VMEM((NBUF * CROW,), jnp.int32),
            pltpu.VMEM((NBUF, CHUNK + 1, EMB), jnp.float32),
            pltpu.SemaphoreType.DMA,
            pltpu.SemaphoreType.DMA,
            pltpu.SemaphoreType.DMA,
        ],
        compiler_params=pltpu.CompilerParams(use_tc_tiling_on_sc=False,
                                             needs_layout_passes=False),
    )
    return kern(ids_flat, weights)


def kernel(input_ids, weights):
    out = _embed(input_ids.reshape(TOTAL), weights)
    return out.reshape(BATCH, SEQ, EMB)
